# baseline (speedup 1.0000x reference)
import functools

import jax
import jax.numpy as jnp
from jax import lax
from jax.experimental import pallas as pl
from jax.experimental.pallas import tpu as pltpu
from jax.experimental.pallas import tpu_sc as plsc

NC = 2
NS = 16


def _scratch(c_chunk, w, n_pad):
    return [
        pltpu.VMEM((c_chunk,), jnp.int32),
        pltpu.VMEM((c_chunk,), jnp.int32),
        pltpu.VMEM((c_chunk, w), jnp.float32),
        pltpu.VMEM((c_chunk,), jnp.int32),
        pltpu.VMEM((c_chunk,), jnp.int32),
        pltpu.VMEM((c_chunk, w), jnp.float32),
        pltpu.VMEM_SHARED((n_pad, w), jnp.float32),
        pltpu.SemaphoreType.DMA,
        pltpu.SemaphoreType.DMA,
        pltpu.SemaphoreType.DMA,
        pltpu.SemaphoreType.DMA,
    ]


def _pipelined_agg(table, src, dst, acc, base, nchunk, c_chunk,
                   iA_s, iA_d, rowsA, iB_s, iB_d, rowsB,
                   gA, gB, siA, siB):
    def sl(j):
        return pl.ds(base + j * c_chunk, c_chunk)

    pltpu.sync_copy(src.at[sl(0)], iA_s)
    pltpu.sync_copy(dst.at[sl(0)], iA_d)
    pltpu.async_copy(table.at[iA_s], rowsA, gA)
    if nchunk > 1:
        pltpu.async_copy(src.at[sl(1)], iB_s, siB)
        pltpu.async_copy(dst.at[sl(1)], iB_d, siB)

    def step(i, m_s, m_d, m_rows, m_g, m_si, o_s, o_d, o_rows, o_g, o_si):
        @pl.when(i + 1 < nchunk)
        def _():
            pltpu.make_async_copy(src.at[sl(i + 1)], o_s, o_si).wait()
            pltpu.make_async_copy(dst.at[sl(i + 1)], o_d, o_si).wait()
        pltpu.make_async_copy(table.at[m_s], m_rows, m_g).wait()

        @pl.when(i + 1 < nchunk)
        def _():
            pltpu.async_copy(table.at[o_s], o_rows, o_g)
        pltpu.sync_copy(m_rows, acc.at[m_d], add=True)

        @pl.when(i + 2 < nchunk)
        def _():
            pltpu.async_copy(src.at[sl(i + 2)], m_s, m_si)
            pltpu.async_copy(dst.at[sl(i + 2)], m_d, m_si)

    def body(i, carry):
        @pl.when(i % 2 == 0)
        def _():
            step(i, iA_s, iA_d, rowsA, gA, siA, iB_s, iB_d, rowsB, gB, siB)

        @pl.when(i % 2 == 1)
        def _():
            step(i, iB_s, iB_d, rowsB, gB, siB, iA_s, iA_d, rowsA, gA, siA)
        return carry

    lax.fori_loop(0, nchunk, body, 0)


def _seg_sum_edge_split(n_pad, e, w, c_chunk):
    nw = NC * NS
    e_per = e // nw
    nchunk = e_per // c_chunk
    rows_per_tile = n_pad // NS
    mesh = plsc.VectorSubcoreMesh(core_axis_name="c", subcore_axis_name="s")

    @functools.partial(
        pl.kernel,
        out_type=jax.ShapeDtypeStruct((NC, n_pad, w), jnp.float32),
        mesh=mesh,
        scratch_types=_scratch(c_chunk, w, n_pad),
        compiler_params=pltpu.CompilerParams(use_tc_tiling_on_sc=False),
    )
    def k(table, src, dst, zeros, out, iA_s, iA_d, rowsA, iB_s, iB_d, rowsB,
          acc, gA, gB, siA, siB):
        c = lax.axis_index("c")
        s = lax.axis_index("s")
        r0 = s * rows_per_tile
        pltpu.sync_copy(zeros, acc.at[pl.ds(r0, rows_per_tile)])
        plsc.subcore_barrier()
        base = (s * NC + c) * e_per
        _pipelined_agg(table, src, dst, acc, base, nchunk, c_chunk,
                       iA_s, iA_d, rowsA, iB_s, iB_d, rowsB, gA, gB, siA, siB)
        plsc.subcore_barrier()
        pltpu.sync_copy(acc.at[pl.ds(r0, rows_per_tile)],
                        out.at[c, pl.ds(r0, rows_per_tile)])

    return k


def _seg_sum_feat_split(n_pad, e, w2, c_chunk):
    e_per = e // NS
    nchunk = e_per // c_chunk
    rows_per_tile = n_pad // NS
    mesh = plsc.VectorSubcoreMesh(core_axis_name="c", subcore_axis_name="s")

    @functools.partial(
        pl.kernel,
        out_type=jax.ShapeDtypeStruct((n_pad, 2 * w2), jnp.float32),
        mesh=mesh,
        scratch_types=_scratch(c_chunk, w2, n_pad),
        compiler_params=pltpu.CompilerParams(use_tc_tiling_on_sc=False),
    )
    def k(table, src, dst, zeros, out, iA_s, iA_d, rowsA, iB_s, iB_d, rowsB,
          acc, gA, gB, siA, siB):
        c = lax.axis_index("c")
        s = lax.axis_index("s")
        r0 = s * rows_per_tile
        pltpu.sync_copy(zeros, acc.at[pl.ds(r0, rows_per_tile)])
        plsc.subcore_barrier()
        base = s * e_per
        _pipelined_agg(table.at[c], src, dst, acc, base, nchunk, c_chunk,
                       iA_s, iA_d, rowsA, iB_s, iB_d, rowsB, gA, gB, siA, siB)
        plsc.subcore_barrier()
        pltpu.sync_copy(acc.at[pl.ds(r0, rows_per_tile)],
                        out.at[pl.ds(r0, rows_per_tile), pl.ds(c * w2, w2)])

    return k


def _full(shape):
    return pl.BlockSpec(shape, lambda i: tuple(0 for _ in shape))


def _rows(b, w):
    return pl.BlockSpec((b, w), lambda i: (i, 0))


def _planes(nc, b, w):
    return pl.BlockSpec((nc, b, w), lambda i: (0, i, 0))


def kernel(x, edge_index, W1_l, b1, W1_r, W2_l, b2, W2_r, W3_l, b3, W3_r,
           W4_l, b4, W4_r):
    n, fin = x.shape
    e = edge_index.shape[1]
    h = W1_l.shape[0]
    fout = W4_l.shape[0]
    hw = h // 2
    w4 = 8
    bsz = 2000
    grid = (n // bsz,)
    f32 = jnp.float32
    n_pad = ((n + NS * 8 - 1) // (NS * 8)) * (NS * 8)

    src = edge_index[0]
    dst = edge_index[1]

    xpad = jnp.concatenate([x, jnp.ones((n, 1), f32),
                            jnp.zeros((n, w4 - fin - 1), f32)], axis=1)
    zeros4 = jnp.zeros((n_pad // NS, w4), f32)
    zeros32 = jnp.zeros((n_pad // NS, hw), f32)

    W1l_t = jnp.zeros((w4, h), f32).at[:fin, :].set(W1_l.T)
    W1r_t = jnp.zeros((w4, h), f32).at[:fin, :].set(W1_r.T)
    W2l_t, W2r_t = W2_l.T, W2_r.T
    W3l_t, W3r_t = W3_l.T, W3_r.T
    W4l_t = jnp.zeros((h, w4), f32).at[:, :fout].set(W4_l.T)
    W4r_t = jnp.zeros((h, w4), f32).at[:, :fout].set(W4_r.T)
    b1r = b1.reshape(1, h)
    b2r = b2.reshape(1, h)
    b3r = b3.reshape(1, h)
    b4r = b4.reshape(1, fout)

    agg1 = _seg_sum_edge_split(n_pad, e, w4, 1000)(xpad, src, dst, zeros4)

    def tc1(xp, parts, w1l, b1_, w1r, w2l, w2r, p2_o, r2_o, ci_o):
        agg = parts[0] + parts[1]
        ci = 1.0 / jnp.maximum(agg[:, fin:fin + 1], 1.0)
        mean = agg * ci
        h1 = jnp.maximum(
            jnp.dot(mean, w1l[...], preferred_element_type=f32) + b1_[...]
            + jnp.dot(xp[...], w1r[...], preferred_element_type=f32), 0.0)
        p2 = jnp.dot(h1, w2l[...], preferred_element_type=f32)
        p2_o[0] = p2[:, :hw]
        p2_o[1] = p2[:, hw:]
        r2_o[...] = jnp.dot(h1, w2r[...], preferred_element_type=f32)
        ci_o[...] = ci

    p2, r2, cntinv = pl.pallas_call(
        tc1,
        grid=grid,
        in_specs=[_rows(bsz, w4), _planes(NC, bsz, w4), _full((w4, h)),
                  _full((1, h)), _full((w4, h)), _full((h, h)),
                  _full((h, h))],
        out_specs=[_planes(NC, bsz, hw), _rows(bsz, h), _rows(bsz, 1)],
        out_shape=[jax.ShapeDtypeStruct((NC, n, hw), f32),
                   jax.ShapeDtypeStruct((n, h), f32),
                   jax.ShapeDtypeStruct((n, 1), f32)],
    )(xpad, agg1, W1l_t, b1r, W1r_t, W2l_t, W2r_t)

    agg2 = _seg_sum_feat_split(n_pad, e, hw, 400)(p2, src, dst, zeros32)

    def tc2(agg, r_in, ci, b_, w3l, w3r, p3_o, r3_o):
        h2 = jnp.maximum(agg[...] * ci[...] + b_[...] + r_in[...], 0.0)
        p3 = jnp.dot(h2, w3l[...], preferred_element_type=f32)
        p3_o[0] = p3[:, :hw]
        p3_o[1] = p3[:, hw:]
        r3_o[...] = jnp.dot(h2, w3r[...], preferred_element_type=f32)

    p3, r3 = pl.pallas_call(
        tc2,
        grid=grid,
        in_specs=[_rows(bsz, h), _rows(bsz, h), _rows(bsz, 1),
                  _full((1, h)), _full((h, h)), _full((h, h))],
        out_specs=[_planes(NC, bsz, hw), _rows(bsz, h)],
        out_shape=[jax.ShapeDtypeStruct((NC, n, hw), f32),
                   jax.ShapeDtypeStruct((n, h), f32)],
    )(agg2, r2, cntinv, b2r, W3l_t, W3r_t)

    agg3 = _seg_sum_feat_split(n_pad, e, hw, 400)(p3, src, dst, zeros32)

    def tc3(agg, r_in, ci, b_, w4l, w4r, p4_o, r4_o):
        h3 = jnp.maximum(agg[...] * ci[...] + b_[...] + r_in[...], 0.0)
        p4_o[...] = jnp.dot(h3, w4l[...], preferred_element_type=f32)
        r4_o[...] = jnp.dot(h3, w4r[...], preferred_element_type=f32)

    p4, r4 = pl.pallas_call(
        tc3,
        grid=grid,
        in_specs=[_rows(bsz, h), _rows(bsz, h), _rows(bsz, 1),
                  _full((1, h)), _full((h, w4)), _full((h, w4))],
        out_specs=[_rows(bsz, w4), _rows(bsz, w4)],
        out_shape=[jax.ShapeDtypeStruct((n, w4), f32),
                   jax.ShapeDtypeStruct((n, w4), f32)],
    )(agg3, r3, cntinv, b3r, W4l_t, W4r_t)

    agg4 = _seg_sum_edge_split(n_pad, e, w4, 1000)(p4, src, dst, zeros4)

    def tc4(parts, r_in, ci, b_, out_o):
        agg = parts[0] + parts[1]
        o2 = agg[:, :fout] * ci[...] + b_[...] + r_in[:, :fout]
        m = jnp.max(o2, axis=1, keepdims=True)
        lse = m + jnp.log(jnp.sum(jnp.exp(o2 - m), axis=1, keepdims=True))
        out_o[...] = o2 - lse

    out = pl.pallas_call(
        tc4,
        grid=grid,
        in_specs=[_planes(NC, bsz, w4), _rows(bsz, w4), _rows(bsz, 1),
                  _full((1, fout))],
        out_specs=[_rows(bsz, fout)],
        out_shape=[jax.ShapeDtypeStruct((n, fout), f32)],
    )(agg4, r4, cntinv, b4r)[0]

    return out


# speedup vs baseline: 1.0627x; 1.0627x over previous
import functools

import jax
import jax.numpy as jnp
from jax import lax
from jax.experimental import pallas as pl
from jax.experimental.pallas import tpu as pltpu
from jax.experimental.pallas import tpu_sc as plsc

NC = 2
NS = 16


def _scratch(c_chunk, w, n_pad, dtype=jnp.float32):
    return [
        pltpu.VMEM((c_chunk,), jnp.int32),
        pltpu.VMEM((c_chunk,), jnp.int32),
        pltpu.VMEM((c_chunk, w), dtype),
        pltpu.VMEM((c_chunk,), jnp.int32),
        pltpu.VMEM((c_chunk,), jnp.int32),
        pltpu.VMEM((c_chunk, w), dtype),
        pltpu.VMEM_SHARED((n_pad, w), dtype),
        pltpu.SemaphoreType.DMA,
        pltpu.SemaphoreType.DMA,
        pltpu.SemaphoreType.DMA,
        pltpu.SemaphoreType.DMA,
    ]


def _pipelined_agg(table, src, dst, acc, base, nchunk, c_chunk,
                   iA_s, iA_d, rowsA, iB_s, iB_d, rowsB,
                   gA, gB, siA, siB):
    def sl(j):
        return pl.ds(base + j * c_chunk, c_chunk)

    pltpu.sync_copy(src.at[sl(0)], iA_s)
    pltpu.sync_copy(dst.at[sl(0)], iA_d)
    pltpu.async_copy(table.at[iA_s], rowsA, gA)
    if nchunk > 1:
        pltpu.async_copy(src.at[sl(1)], iB_s, siB)
        pltpu.async_copy(dst.at[sl(1)], iB_d, siB)

    def step(i, m_s, m_d, m_rows, m_g, m_si, o_s, o_d, o_rows, o_g, o_si):
        @pl.when(i + 1 < nchunk)
        def _():
            pltpu.make_async_copy(src.at[sl(i + 1)], o_s, o_si).wait()
            pltpu.make_async_copy(dst.at[sl(i + 1)], o_d, o_si).wait()
        pltpu.make_async_copy(table.at[m_s], m_rows, m_g).wait()

        @pl.when(i + 1 < nchunk)
        def _():
            pltpu.async_copy(table.at[o_s], o_rows, o_g)
        pltpu.sync_copy(m_rows, acc.at[m_d], add=True)

        @pl.when(i + 2 < nchunk)
        def _():
            pltpu.async_copy(src.at[sl(i + 2)], m_s, m_si)
            pltpu.async_copy(dst.at[sl(i + 2)], m_d, m_si)

    def body(i, carry):
        @pl.when(i % 2 == 0)
        def _():
            step(i, iA_s, iA_d, rowsA, gA, siA, iB_s, iB_d, rowsB, gB, siB)

        @pl.when(i % 2 == 1)
        def _():
            step(i, iB_s, iB_d, rowsB, gB, siB, iA_s, iA_d, rowsA, gA, siA)
        return carry

    lax.fori_loop(0, nchunk, body, 0)


def _seg_sum_edge_split(n_pad, e, w, c_chunk):
    nw = NC * NS
    e_per = e // nw
    nchunk = e_per // c_chunk
    rows_per_tile = n_pad // NS
    mesh = plsc.VectorSubcoreMesh(core_axis_name="c", subcore_axis_name="s")

    @functools.partial(
        pl.kernel,
        out_type=jax.ShapeDtypeStruct((NC, n_pad, w), jnp.float32),
        mesh=mesh,
        scratch_types=_scratch(c_chunk, w, n_pad),
        compiler_params=pltpu.CompilerParams(use_tc_tiling_on_sc=False),
    )
    def k(table, src, dst, zeros, out, iA_s, iA_d, rowsA, iB_s, iB_d, rowsB,
          acc, gA, gB, siA, siB):
        c = lax.axis_index("c")
        s = lax.axis_index("s")
        r0 = s * rows_per_tile
        pltpu.sync_copy(zeros, acc.at[pl.ds(r0, rows_per_tile)])
        plsc.subcore_barrier()
        base = (s * NC + c) * e_per
        _pipelined_agg(table, src, dst, acc, base, nchunk, c_chunk,
                       iA_s, iA_d, rowsA, iB_s, iB_d, rowsB, gA, gB, siA, siB)
        plsc.subcore_barrier()
        pltpu.sync_copy(acc.at[pl.ds(r0, rows_per_tile)],
                        out.at[c, pl.ds(r0, rows_per_tile)])

    return k


def _seg_sum_feat_split(n_pad, e, w2, c_chunk, dtype=jnp.float32):
    e_per = e // NS
    nchunk = e_per // c_chunk
    rows_per_tile = n_pad // NS
    mesh = plsc.VectorSubcoreMesh(core_axis_name="c", subcore_axis_name="s")

    @functools.partial(
        pl.kernel,
        out_type=jax.ShapeDtypeStruct((n_pad, 2 * w2), dtype),
        mesh=mesh,
        scratch_types=_scratch(c_chunk, w2, n_pad, dtype),
        compiler_params=pltpu.CompilerParams(use_tc_tiling_on_sc=False),
    )
    def k(table, src, dst, zeros, out, iA_s, iA_d, rowsA, iB_s, iB_d, rowsB,
          acc, gA, gB, siA, siB):
        c = lax.axis_index("c")
        s = lax.axis_index("s")
        r0 = s * rows_per_tile
        pltpu.sync_copy(zeros, acc.at[pl.ds(r0, rows_per_tile)])
        plsc.subcore_barrier()
        base = s * e_per
        _pipelined_agg(table.at[c], src, dst, acc, base, nchunk, c_chunk,
                       iA_s, iA_d, rowsA, iB_s, iB_d, rowsB, gA, gB, siA, siB)
        plsc.subcore_barrier()
        pltpu.sync_copy(acc.at[pl.ds(r0, rows_per_tile)],
                        out.at[pl.ds(r0, rows_per_tile), pl.ds(c * w2, w2)])

    return k


def _full(shape):
    return pl.BlockSpec(shape, lambda i: tuple(0 for _ in shape))


def _rows(b, w):
    return pl.BlockSpec((b, w), lambda i: (i, 0))


def _planes(nc, b, w):
    return pl.BlockSpec((nc, b, w), lambda i: (0, i, 0))


def kernel(x, edge_index, W1_l, b1, W1_r, W2_l, b2, W2_r, W3_l, b3, W3_r,
           W4_l, b4, W4_r):
    n, fin = x.shape
    e = edge_index.shape[1]
    h = W1_l.shape[0]
    fout = W4_l.shape[0]
    hw = h // 2
    w4 = 8
    bsz = 2000
    grid = (n // bsz,)
    f32 = jnp.float32
    bf16 = jnp.bfloat16
    n_pad = ((n + NS * 8 - 1) // (NS * 8)) * (NS * 8)

    src = edge_index[0]
    dst = edge_index[1]

    xpad = jnp.concatenate([x, jnp.ones((n, 1), f32),
                            jnp.zeros((n, w4 - fin - 1), f32)], axis=1)
    zeros4 = jnp.zeros((n_pad // NS, w4), f32)
    zeros32 = jnp.zeros((n_pad // NS, hw), bf16)

    W1l_t = jnp.zeros((w4, h), f32).at[:fin, :].set(W1_l.T)
    W1r_t = jnp.zeros((w4, h), f32).at[:fin, :].set(W1_r.T)
    W2l_t, W2r_t = W2_l.T, W2_r.T
    W3l_t, W3r_t = W3_l.T, W3_r.T
    W4l_t = jnp.zeros((h, w4), f32).at[:, :fout].set(W4_l.T)
    W4r_t = jnp.zeros((h, w4), f32).at[:, :fout].set(W4_r.T)
    b1r = b1.reshape(1, h)
    b2r = b2.reshape(1, h)
    b3r = b3.reshape(1, h)
    b4r = b4.reshape(1, fout)

    agg1 = _seg_sum_edge_split(n_pad, e, w4, 1000)(xpad, src, dst, zeros4)

    def tc1(xp, parts, w1l, b1_, w1r, w2l, w2r, p2_o, r2_o, ci_o):
        agg = parts[0] + parts[1]
        ci = 1.0 / jnp.maximum(agg[:, fin:fin + 1], 1.0)
        mean = agg * ci
        h1 = jnp.maximum(
            jnp.dot(mean, w1l[...], preferred_element_type=f32) + b1_[...]
            + jnp.dot(xp[...], w1r[...], preferred_element_type=f32), 0.0)
        p2 = jnp.dot(h1, w2l[...], preferred_element_type=f32)
        p2_o[0] = p2[:, :hw].astype(jnp.bfloat16)
        p2_o[1] = p2[:, hw:].astype(jnp.bfloat16)
        r2_o[...] = jnp.dot(h1, w2r[...], preferred_element_type=f32)
        ci_o[...] = ci

    p2, r2, cntinv = pl.pallas_call(
        tc1,
        grid=grid,
        in_specs=[_rows(bsz, w4), _planes(NC, bsz, w4), _full((w4, h)),
                  _full((1, h)), _full((w4, h)), _full((h, h)),
                  _full((h, h))],
        out_specs=[_planes(NC, bsz, hw), _rows(bsz, h), _rows(bsz, 1)],
        out_shape=[jax.ShapeDtypeStruct((NC, n, hw), bf16),
                   jax.ShapeDtypeStruct((n, h), f32),
                   jax.ShapeDtypeStruct((n, 1), f32)],
    )(xpad, agg1, W1l_t, b1r, W1r_t, W2l_t, W2r_t)

    agg2 = _seg_sum_feat_split(n_pad, e, hw, 1000, bf16)(p2, src, dst, zeros32)

    def tc2(agg, r_in, ci, b_, w3l, w3r, p3_o, r3_o):
        h2 = jnp.maximum(agg[...].astype(jnp.float32) * ci[...] + b_[...]
                         + r_in[...], 0.0)
        p3 = jnp.dot(h2, w3l[...], preferred_element_type=f32)
        p3_o[0] = p3[:, :hw].astype(jnp.bfloat16)
        p3_o[1] = p3[:, hw:].astype(jnp.bfloat16)
        r3_o[...] = jnp.dot(h2, w3r[...], preferred_element_type=f32)

    p3, r3 = pl.pallas_call(
        tc2,
        grid=grid,
        in_specs=[_rows(bsz, h), _rows(bsz, h), _rows(bsz, 1),
                  _full((1, h)), _full((h, h)), _full((h, h))],
        out_specs=[_planes(NC, bsz, hw), _rows(bsz, h)],
        out_shape=[jax.ShapeDtypeStruct((NC, n, hw), bf16),
                   jax.ShapeDtypeStruct((n, h), f32)],
    )(agg2, r2, cntinv, b2r, W3l_t, W3r_t)

    agg3 = _seg_sum_feat_split(n_pad, e, hw, 1000, bf16)(p3, src, dst, zeros32)

    def tc3(agg, r_in, ci, b_, w4l, w4r, p4_o, r4_o):
        h3 = jnp.maximum(agg[...].astype(jnp.float32) * ci[...] + b_[...]
                         + r_in[...], 0.0)
        p4_o[...] = jnp.dot(h3, w4l[...], preferred_element_type=f32)
        r4_o[...] = jnp.dot(h3, w4r[...], preferred_element_type=f32)

    p4, r4 = pl.pallas_call(
        tc3,
        grid=grid,
        in_specs=[_rows(bsz, h), _rows(bsz, h), _rows(bsz, 1),
                  _full((1, h)), _full((h, w4)), _full((h, w4))],
        out_specs=[_rows(bsz, w4), _rows(bsz, w4)],
        out_shape=[jax.ShapeDtypeStruct((n, w4), f32),
                   jax.ShapeDtypeStruct((n, w4), f32)],
    )(agg3, r3, cntinv, b3r, W4l_t, W4r_t)

    agg4 = _seg_sum_edge_split(n_pad, e, w4, 1000)(p4, src, dst, zeros4)

    def tc4(parts, r_in, ci, b_, out_o):
        agg = parts[0] + parts[1]
        o2 = agg[:, :fout] * ci[...] + b_[...] + r_in[:, :fout]
        m = jnp.max(o2, axis=1, keepdims=True)
        lse = m + jnp.log(jnp.sum(jnp.exp(o2 - m), axis=1, keepdims=True))
        out_o[...] = o2 - lse

    out = pl.pallas_call(
        tc4,
        grid=grid,
        in_specs=[_planes(NC, bsz, w4), _rows(bsz, w4), _rows(bsz, 1),
                  _full((1, fout))],
        out_specs=[_rows(bsz, fout)],
        out_shape=[jax.ShapeDtypeStruct((n, fout), f32)],
    )(agg4, r4, cntinv, b4r)[0]

    return out


# speedup vs baseline: 1.1298x; 1.0631x over previous
import functools

import jax
import jax.numpy as jnp
from jax import lax
from jax.experimental import pallas as pl
from jax.experimental.pallas import tpu as pltpu
from jax.experimental.pallas import tpu_sc as plsc

NC = 2
NS = 16


def _scratch(c_chunk, w, n_pad, dtype=jnp.float32):
    return [
        pltpu.VMEM((c_chunk,), jnp.int32),
        pltpu.VMEM((c_chunk,), jnp.int32),
        pltpu.VMEM((c_chunk, w), dtype),
        pltpu.VMEM((c_chunk,), jnp.int32),
        pltpu.VMEM((c_chunk,), jnp.int32),
        pltpu.VMEM((c_chunk, w), dtype),
        pltpu.VMEM_SHARED((n_pad, w), dtype),
        pltpu.SemaphoreType.DMA,
        pltpu.SemaphoreType.DMA,
        pltpu.SemaphoreType.DMA,
        pltpu.SemaphoreType.DMA,
    ]


def _pipelined_agg(table, src, dst, acc, base, nchunk, c_chunk,
                   iA_s, iA_d, rowsA, iB_s, iB_d, rowsB,
                   gA, gB, siA, siB):
    def sl(j):
        return pl.ds(base + j * c_chunk, c_chunk)

    pltpu.sync_copy(src.at[sl(0)], iA_s)
    pltpu.sync_copy(dst.at[sl(0)], iA_d)
    pltpu.async_copy(table.at[iA_s], rowsA, gA)
    if nchunk > 1:
        pltpu.async_copy(src.at[sl(1)], iB_s, siB)
        pltpu.async_copy(dst.at[sl(1)], iB_d, siB)

    def step(i, m_s, m_d, m_rows, m_g, m_si, o_s, o_d, o_rows, o_g, o_si):
        @pl.when(i + 1 < nchunk)
        def _():
            pltpu.make_async_copy(src.at[sl(i + 1)], o_s, o_si).wait()
            pltpu.make_async_copy(dst.at[sl(i + 1)], o_d, o_si).wait()
        pltpu.make_async_copy(table.at[m_s], m_rows, m_g).wait()

        @pl.when(i + 1 < nchunk)
        def _():
            pltpu.async_copy(table.at[o_s], o_rows, o_g)
        pltpu.sync_copy(m_rows, acc.at[m_d], add=True)

        @pl.when(i + 2 < nchunk)
        def _():
            pltpu.async_copy(src.at[sl(i + 2)], m_s, m_si)
            pltpu.async_copy(dst.at[sl(i + 2)], m_d, m_si)

    def body(i, carry):
        @pl.when(i % 2 == 0)
        def _():
            step(i, iA_s, iA_d, rowsA, gA, siA, iB_s, iB_d, rowsB, gB, siB)

        @pl.when(i % 2 == 1)
        def _():
            step(i, iB_s, iB_d, rowsB, gB, siB, iA_s, iA_d, rowsA, gA, siA)
        return carry

    lax.fori_loop(0, nchunk, body, 0)


def _seg_sum_edge_split(n_pad, e, w, c_chunk):
    nw = NC * NS
    e_per = e // nw
    nchunk = e_per // c_chunk
    rows_per_tile = n_pad // NS
    mesh = plsc.VectorSubcoreMesh(core_axis_name="c", subcore_axis_name="s")

    @functools.partial(
        pl.kernel,
        out_type=jax.ShapeDtypeStruct((NC, n_pad, w), jnp.float32),
        mesh=mesh,
        scratch_types=_scratch(c_chunk, w, n_pad),
        compiler_params=pltpu.CompilerParams(use_tc_tiling_on_sc=False),
    )
    def k(table, src, dst, zeros, out, iA_s, iA_d, rowsA, iB_s, iB_d, rowsB,
          acc, gA, gB, siA, siB):
        c = lax.axis_index("c")
        s = lax.axis_index("s")
        r0 = s * rows_per_tile
        pltpu.sync_copy(zeros, acc.at[pl.ds(r0, rows_per_tile)])
        plsc.subcore_barrier()
        base = (s * NC + c) * e_per
        _pipelined_agg(table, src, dst, acc, base, nchunk, c_chunk,
                       iA_s, iA_d, rowsA, iB_s, iB_d, rowsB, gA, gB, siA, siB)
        plsc.subcore_barrier()
        pltpu.sync_copy(acc.at[pl.ds(r0, rows_per_tile)],
                        out.at[c, pl.ds(r0, rows_per_tile)])

    return k


def _seg_sum_feat_split(n_pad, e, w2, c_chunk, dtype=jnp.float32):
    e_per = e // NS
    nchunk = e_per // c_chunk
    rows_per_tile = n_pad // NS
    mesh = plsc.VectorSubcoreMesh(core_axis_name="c", subcore_axis_name="s")

    @functools.partial(
        pl.kernel,
        out_type=jax.ShapeDtypeStruct((n_pad, 2 * w2), dtype),
        mesh=mesh,
        scratch_types=_scratch(c_chunk, w2, n_pad, dtype),
        compiler_params=pltpu.CompilerParams(use_tc_tiling_on_sc=False),
    )
    def k(table, src, dst, zeros, out, iA_s, iA_d, rowsA, iB_s, iB_d, rowsB,
          acc, gA, gB, siA, siB):
        c = lax.axis_index("c")
        s = lax.axis_index("s")
        r0 = s * rows_per_tile
        pltpu.sync_copy(zeros, acc.at[pl.ds(r0, rows_per_tile)])
        plsc.subcore_barrier()
        base = s * e_per
        _pipelined_agg(table.at[c], src, dst, acc, base, nchunk, c_chunk,
                       iA_s, iA_d, rowsA, iB_s, iB_d, rowsB, gA, gB, siA, siB)
        plsc.subcore_barrier()
        pltpu.sync_copy(acc.at[pl.ds(r0, rows_per_tile)],
                        out.at[pl.ds(r0, rows_per_tile), pl.ds(c * w2, w2)])

    return k


def _full(shape):
    return pl.BlockSpec(shape, lambda i: tuple(0 for _ in shape))


def _rows(b, w):
    return pl.BlockSpec((b, w), lambda i: (i, 0))


def _planes(nc, b, w):
    return pl.BlockSpec((nc, b, w), lambda i: (0, i, 0))


def kernel(x, edge_index, W1_l, b1, W1_r, W2_l, b2, W2_r, W3_l, b3, W3_r,
           W4_l, b4, W4_r):
    n, fin = x.shape
    e = edge_index.shape[1]
    h = W1_l.shape[0]
    fout = W4_l.shape[0]
    hw = h // 2
    w4 = 8
    bsz = 2000
    grid = (n // bsz,)
    f32 = jnp.float32
    bf16 = jnp.bfloat16
    n_pad = ((n + NS * 8 - 1) // (NS * 8)) * (NS * 8)

    src = edge_index[0]
    dst = edge_index[1]

    xpad = jnp.concatenate([x, jnp.ones((n, 1), f32),
                            jnp.zeros((n, w4 - fin - 1), f32)], axis=1)
    zeros4 = jnp.zeros((n_pad // NS, w4), f32)
    zeros32 = jnp.zeros((n_pad // NS, hw), bf16)

    W1l_t = jnp.zeros((w4, h), f32).at[:fin, :].set(W1_l.T)
    W1r_t = jnp.zeros((w4, h), f32).at[:fin, :].set(W1_r.T)
    W2l_t, W2r_t = W2_l.T, W2_r.T
    W3l_t, W3r_t = W3_l.T, W3_r.T
    W4l_t = jnp.zeros((h, w4), f32).at[:, :fout].set(W4_l.T)
    W4r_t = jnp.zeros((h, w4), f32).at[:, :fout].set(W4_r.T)
    b1r = b1.reshape(1, h)
    b2r = b2.reshape(1, h)
    b3r = b3.reshape(1, h)
    b4r = b4.reshape(1, fout)

    agg1 = _seg_sum_edge_split(n_pad, e, w4, 5000)(xpad, src, dst, zeros4)

    def tc1(xp, parts, w1l, b1_, w1r, w2l, w2r, p2_o, r2_o, ci_o):
        agg = parts[0] + parts[1]
        ci = 1.0 / jnp.maximum(agg[:, fin:fin + 1], 1.0)
        mean = agg * ci
        h1 = jnp.maximum(
            jnp.dot(mean, w1l[...], preferred_element_type=f32) + b1_[...]
            + jnp.dot(xp[...], w1r[...], preferred_element_type=f32), 0.0)
        p2 = jnp.dot(h1, w2l[...], preferred_element_type=f32)
        p2_o[0] = p2[:, :hw].astype(jnp.bfloat16)
        p2_o[1] = p2[:, hw:].astype(jnp.bfloat16)
        r2_o[...] = jnp.dot(h1, w2r[...], preferred_element_type=f32)
        ci_o[...] = ci

    p2, r2, cntinv = pl.pallas_call(
        tc1,
        grid=grid,
        in_specs=[_rows(bsz, w4), _planes(NC, bsz, w4), _full((w4, h)),
                  _full((1, h)), _full((w4, h)), _full((h, h)),
                  _full((h, h))],
        out_specs=[_planes(NC, bsz, hw), _rows(bsz, h), _rows(bsz, 1)],
        out_shape=[jax.ShapeDtypeStruct((NC, n, hw), bf16),
                   jax.ShapeDtypeStruct((n, h), f32),
                   jax.ShapeDtypeStruct((n, 1), f32)],
    )(xpad, agg1, W1l_t, b1r, W1r_t, W2l_t, W2r_t)

    agg2 = _seg_sum_feat_split(n_pad, e, hw, 2000, bf16)(p2, src, dst, zeros32)

    def tc2(agg, r_in, ci, b_, w3l, w3r, p3_o, r3_o):
        h2 = jnp.maximum(agg[...].astype(jnp.float32) * ci[...] + b_[...]
                         + r_in[...], 0.0)
        p3 = jnp.dot(h2, w3l[...], preferred_element_type=f32)
        p3_o[0] = p3[:, :hw].astype(jnp.bfloat16)
        p3_o[1] = p3[:, hw:].astype(jnp.bfloat16)
        r3_o[...] = jnp.dot(h2, w3r[...], preferred_element_type=f32)

    p3, r3 = pl.pallas_call(
        tc2,
        grid=grid,
        in_specs=[_rows(bsz, h), _rows(bsz, h), _rows(bsz, 1),
                  _full((1, h)), _full((h, h)), _full((h, h))],
        out_specs=[_planes(NC, bsz, hw), _rows(bsz, h)],
        out_shape=[jax.ShapeDtypeStruct((NC, n, hw), bf16),
                   jax.ShapeDtypeStruct((n, h), f32)],
    )(agg2, r2, cntinv, b2r, W3l_t, W3r_t)

    agg3 = _seg_sum_feat_split(n_pad, e, hw, 2000, bf16)(p3, src, dst, zeros32)

    def tc3(agg, r_in, ci, b_, w4l, w4r, p4_o, r4_o):
        h3 = jnp.maximum(agg[...].astype(jnp.float32) * ci[...] + b_[...]
                         + r_in[...], 0.0)
        p4_o[...] = jnp.dot(h3, w4l[...], preferred_element_type=f32)
        r4_o[...] = jnp.dot(h3, w4r[...], preferred_element_type=f32)

    p4, r4 = pl.pallas_call(
        tc3,
        grid=grid,
        in_specs=[_rows(bsz, h), _rows(bsz, h), _rows(bsz, 1),
                  _full((1, h)), _full((h, w4)), _full((h, w4))],
        out_specs=[_rows(bsz, w4), _rows(bsz, w4)],
        out_shape=[jax.ShapeDtypeStruct((n, w4), f32),
                   jax.ShapeDtypeStruct((n, w4), f32)],
    )(agg3, r3, cntinv, b3r, W4l_t, W4r_t)

    agg4 = _seg_sum_edge_split(n_pad, e, w4, 5000)(p4, src, dst, zeros4)

    def tc4(parts, r_in, ci, b_, out_o):
        agg = parts[0] + parts[1]
        o2 = agg[:, :fout] * ci[...] + b_[...] + r_in[:, :fout]
        m = jnp.max(o2, axis=1, keepdims=True)
        lse = m + jnp.log(jnp.sum(jnp.exp(o2 - m), axis=1, keepdims=True))
        out_o[...] = o2 - lse

    out = pl.pallas_call(
        tc4,
        grid=grid,
        in_specs=[_planes(NC, bsz, w4), _rows(bsz, w4), _rows(bsz, 1),
                  _full((1, fout))],
        out_specs=[_rows(bsz, fout)],
        out_shape=[jax.ShapeDtypeStruct((n, fout), f32)],
    )(agg4, r4, cntinv, b4r)[0]

    return out
